# Initial kernel scaffold; baseline (speedup 1.0000x reference)
import functools

import jax
import jax.numpy as jnp
from jax import lax
from jax.experimental import pallas as pl
from jax.experimental.pallas import tpu as pltpu
from jax.experimental.pallas import tpu_sc as plsc

_D = 32
_NW = 32
_CHUNK = 1664


@functools.cache
def _make_gather(n_rows: int):
    assert n_rows % (_NW * _CHUNK) == 0
    b_per_w = n_rows // _NW
    nchunk = b_per_w // _CHUNK
    mesh = plsc.VectorSubcoreMesh(core_axis_name="c", subcore_axis_name="s")

    @functools.partial(
        pl.kernel,
        mesh=mesh,
        out_type=jax.ShapeDtypeStruct((n_rows, _D), jnp.float32),
        scratch_types=[
            pltpu.VMEM((nchunk, _CHUNK), jnp.int32),
            pltpu.VMEM((_CHUNK, _D), jnp.float32),
            pltpu.SemaphoreType.DMA,
        ],
    )
    def gather(idx_hbm, table_hbm, out_hbm, idx_v, rows_v, gsem):
        wid = lax.axis_index("s") * 2 + lax.axis_index("c")
        base = wid * b_per_w
        pltpu.sync_copy(idx_hbm.at[wid], idx_v)
        for c in range(nchunk):
            pltpu.async_copy(table_hbm.at[idx_v.at[c]], rows_v, gsem).wait()
            pltpu.sync_copy(rows_v, out_hbm.at[pl.ds(base + c * _CHUNK, _CHUNK)])

    return gather


def kernel(x, table):
    b, f = x.shape
    n = b * f
    idx = x.reshape(_NW, n // (_NW * _CHUNK), _CHUNK).astype(jnp.int32)
    out = _make_gather(n)(idx, table)
    return out.reshape(b, f, _D)


# speedup vs baseline: 1.5678x; 1.5678x over previous
import functools

import jax
import jax.numpy as jnp
from jax import lax
from jax.experimental import pallas as pl
from jax.experimental.pallas import tpu as pltpu
from jax.experimental.pallas import tpu_sc as plsc

_D = 32
_NW = 32
_CHUNK = 1664


@functools.cache
def _make_gather(n_rows: int):
    assert n_rows % (_NW * _CHUNK) == 0
    b_per_w = n_rows // _NW
    nchunk = b_per_w // _CHUNK
    mesh = plsc.VectorSubcoreMesh(core_axis_name="c", subcore_axis_name="s")

    @functools.partial(
        pl.kernel,
        mesh=mesh,
        out_type=jax.ShapeDtypeStruct((n_rows, _D), jnp.float32),
        compiler_params=pltpu.CompilerParams(use_tc_tiling_on_sc=False),
        scratch_types=[
            pltpu.VMEM((nchunk, _CHUNK), jnp.int32),
            pltpu.VMEM((_CHUNK, _D), jnp.float32),
            pltpu.SemaphoreType.DMA,
        ],
    )
    def gather(idx_hbm, table_hbm, out_hbm, idx_v, rows_v, gsem):
        wid = lax.axis_index("s") * 2 + lax.axis_index("c")
        base = wid * b_per_w
        pltpu.sync_copy(idx_hbm.at[wid], idx_v)
        for c in range(nchunk):
            pltpu.async_copy(table_hbm.at[idx_v.at[c]], rows_v, gsem).wait()
            pltpu.sync_copy(rows_v, out_hbm.at[pl.ds(base + c * _CHUNK, _CHUNK)])

    return gather


def kernel(x, table):
    b, f = x.shape
    n = b * f
    idx = x.reshape(_NW, n // (_NW * _CHUNK), _CHUNK).astype(jnp.int32)
    out = _make_gather(n)(idx, table)
    return out.reshape(b, f, _D)


# speedup vs baseline: 1.5807x; 1.0082x over previous
import functools

import jax
import jax.numpy as jnp
from jax import lax
from jax.experimental import pallas as pl
from jax.experimental.pallas import tpu as pltpu
from jax.experimental.pallas import tpu_sc as plsc

_D = 32
_NW = 32
_CHUNK = 1664


@functools.cache
def _make_gather(n_rows: int):
    assert n_rows % (_NW * _CHUNK) == 0
    b_per_w = n_rows // _NW
    nchunk = b_per_w // _CHUNK
    mesh = plsc.VectorSubcoreMesh(core_axis_name="c", subcore_axis_name="s")

    @functools.partial(
        pl.kernel,
        mesh=mesh,
        out_type=jax.ShapeDtypeStruct((n_rows, _D), jnp.float32),
        compiler_params=pltpu.CompilerParams(use_tc_tiling_on_sc=False),
        scratch_types=[
            pltpu.VMEM((nchunk, _CHUNK), jnp.int32),
            pltpu.VMEM((2, _CHUNK, _D), jnp.float32),
            pltpu.SemaphoreType.DMA,
            pltpu.SemaphoreType.DMA,
            pltpu.SemaphoreType.DMA,
            pltpu.SemaphoreType.DMA,
        ],
    )
    def gather(idx_hbm, table_hbm, out_hbm, idx_v, rows_v, g0, g1, s0, s1):
        wid = lax.axis_index("s") * 2 + lax.axis_index("c")
        base = wid * b_per_w
        gsem = (g0, g1)
        ssem = (s0, s1)
        pltpu.sync_copy(idx_hbm.at[wid], idx_v)

        def start_gather(c, buf):
            return pltpu.async_copy(
                table_hbm.at[idx_v.at[c]], rows_v.at[buf], gsem[buf])

        def start_store(c, buf):
            return pltpu.async_copy(
                rows_v.at[buf],
                out_hbm.at[pl.ds(base + c * _CHUNK, _CHUNK)],
                ssem[buf])

        g = [None, None]
        s = [None, None]
        g[0] = start_gather(0, 0)
        for c in range(nchunk):
            buf = c & 1
            if c + 1 < nchunk:
                if s[1 - buf] is not None:
                    s[1 - buf].wait()
                g[1 - buf] = start_gather(c + 1, 1 - buf)
            g[buf].wait()
            s[buf] = start_store(c, buf)
        s[(nchunk - 2) & 1].wait()
        s[(nchunk - 1) & 1].wait()

    return gather


def kernel(x, table):
    b, f = x.shape
    n = b * f
    idx = x.reshape(_NW, n // (_NW * _CHUNK), _CHUNK).astype(jnp.int32)
    out = _make_gather(n)(idx, table)
    return out.reshape(b, f, _D)
